# baseline (speedup 1.0000x reference)
import functools

import jax
import jax.numpy as jnp
from jax import lax
from jax.experimental import pallas as pl
from jax.experimental.pallas import tpu as pltpu
from jax.experimental.pallas import tpu_sc as plsc

EMB = 64
WIDE = 2 * EMB
IDX_CHUNK = 128
PBLK = 4096
TC_BLOCKS = 88


def _pair_body(a_ref, b_ref, out_ref):
    at = a_ref[...].astype(jnp.bfloat16).T.astype(jnp.float32)
    bt = b_ref[...].astype(jnp.bfloat16).T.astype(jnp.float32)
    out_ref[...] = jnp.concatenate([at, bt], axis=1)


def _pair_table(tt, nb):
    return pl.pallas_call(
        _pair_body,
        grid=(nb,),
        in_specs=[
            pl.BlockSpec((EMB, PBLK), lambda i: (0, 2 * i)),
            pl.BlockSpec((EMB, PBLK), lambda i: (0, 2 * i + 1)),
        ],
        out_specs=pl.BlockSpec((PBLK, WIDE), lambda i: (i, 0)),
        out_shape=jax.ShapeDtypeStruct((nb * PBLK, WIDE), jnp.float32),
    )(tt, tt)


def _make_gather(B):
    info = plsc.get_sparse_core_info()
    NC, NS = info.num_cores, info.num_subcores
    NW = NC * NS
    b_per_w = B // NW
    n_chunks = b_per_w // IDX_CHUNK
    mesh = plsc.VectorSubcoreMesh(core_axis_name="c", subcore_axis_name="s")

    @functools.partial(
        pl.kernel,
        mesh=mesh,
        out_type=[jax.ShapeDtypeStruct((B, WIDE), jnp.float32)
                  for _ in range(3)],
        scratch_types=(
            [pltpu.VMEM((b_per_w,), jnp.int32) for _ in range(3)]
            + [pltpu.VMEM((2, IDX_CHUNK, WIDE), jnp.float32)
               for _ in range(3)]
            + [pltpu.SemaphoreType.DMA for _ in range(6)]
        ),
    )
    def gather_k(ia_h, ib_h, im_h, ta_h, tb_h, tm_h, oa, ob, om,
                 ia_v, ib_v, im_v, abuf, bbuf, mbuf, *sems):
        wid = lax.axis_index("s") * NC + lax.axis_index("c")
        base = wid * b_per_w
        idxs = (ia_v, ib_v, im_v)
        tabs = (ta_h, tb_h, tm_h)
        outs = (oa, ob, om)
        bufs = (abuf, bbuf, mbuf)
        for ih, iv in zip((ia_h, ib_h, im_h), idxs):
            pltpu.sync_copy(ih.at[pl.ds(base, b_per_w)], iv)

        def start(c):
            slot = c % 2
            sl = pl.ds(c * IDX_CHUNK, IDX_CHUNK)
            return [pltpu.async_copy(tabs[t].at[idxs[t].at[sl]],
                                     bufs[t].at[slot], sems[2 * t + slot])
                    for t in range(3)]

        pend = start(0)
        for c in range(n_chunks):
            slot = c % 2
            cur = pend
            if c + 1 < n_chunks:
                pend = start(c + 1)
            osl = pl.ds(base + c * IDX_CHUNK, IDX_CHUNK)
            for t in range(3):
                cur[t].wait()
                pltpu.sync_copy(bufs[t].at[slot], outs[t].at[osl])

    return gather_k


def _half(w, p_ref):
    return w[:, :EMB] + p_ref[...] * (w[:, EMB:] - w[:, :EMB])


def _mlp_body(wa_ref, wb_ref, wm_ref, pa_ref, pb_ref, pm_ref, sb_ref,
              w1u_ref, w1m_ref, b1_ref, w2_ref, b2_ref, w3_ref, b3_ref,
              out_ref):
    uea = _half(wa_ref[...], pa_ref)
    ueb = _half(wb_ref[...], pb_ref)
    ue = uea + sb_ref[...] * (ueb - uea)
    me = _half(wm_ref[...], pm_ref)
    h = jnp.dot(ue, w1u_ref[...], preferred_element_type=jnp.float32)
    h = h + jnp.dot(me, w1m_ref[...], preferred_element_type=jnp.float32)
    h = jnp.maximum(h + b1_ref[...], 0.0)
    h = jnp.dot(h, w2_ref[...], preferred_element_type=jnp.float32)
    h = jnp.maximum(h + b2_ref[...], 0.0)
    o = jnp.sum(h * w3_ref[...], axis=1) + b3_ref[0, 0]
    out_ref[...] = o[None, :]


def _mlp(wa, wb, wm, pa, pb, pm, sb, w1u, w1m, b1, w2, b2, w3row, b3,
         blk=2048):
    B = wa.shape[0]
    grid = (B // blk,)
    const = lambda i: (0, 0)
    row = lambda i: (i, 0)
    return pl.pallas_call(
        _mlp_body,
        grid=grid,
        in_specs=[
            pl.BlockSpec((blk, WIDE), row),
            pl.BlockSpec((blk, WIDE), row),
            pl.BlockSpec((blk, WIDE), row),
            pl.BlockSpec((blk, 1), row),
            pl.BlockSpec((blk, 1), row),
            pl.BlockSpec((blk, 1), row),
            pl.BlockSpec((blk, 1), row),
            pl.BlockSpec((EMB, 128), const),
            pl.BlockSpec((EMB, 128), const),
            pl.BlockSpec((1, 128), const),
            pl.BlockSpec((128, 64), const),
            pl.BlockSpec((1, 64), const),
            pl.BlockSpec((1, 64), const),
            pl.BlockSpec((1, 1), const),
        ],
        out_specs=pl.BlockSpec((1, blk), lambda i: (0, i)),
        out_shape=jax.ShapeDtypeStruct((1, B), jnp.float32),
    )(wa, wb, wm, pa, pb, pm, sb, w1u, w1m, b1, w2, b2, w3row, b3)


def kernel(users, movies, user_table, movie_table, W1, b1, W2, b2, W3, b3):
    B = users.shape[0]
    nu = user_table.shape[0]
    nm = movie_table.shape[0]
    split = TC_BLOCKS * 2 * PBLK
    u32 = users.astype(jnp.int32)
    m32 = movies.astype(jnp.int32)

    uta = _pair_table(user_table.T, TC_BLOCKS)
    utb = user_table[split:].reshape(-1, WIDE)
    mt2 = jnp.concatenate(
        [movie_table[: nm // 2], movie_table[nm // 2 :]], axis=1)

    sel_b = u32 >= split
    rowa = jnp.where(sel_b, 0, (u32 // (2 * PBLK)) * PBLK + (u32 & (PBLK - 1)))
    ub = jnp.maximum(u32 - split, 0)
    rowb = ub >> 1
    rowm = m32 % (nm // 2)

    gather_k = _make_gather(B)
    wa, wb, wm = gather_k(rowa, rowb, rowm, uta, utb, mt2)

    f32col = lambda x: x.astype(jnp.float32).reshape(B, 1)
    pa = f32col((u32 & (2 * PBLK - 1)) >= PBLK)
    pb = f32col(ub & 1)
    pm = f32col(m32 >= nm // 2)
    sb = f32col(sel_b)
    out = _mlp(wa, wb, wm, pa, pb, pm, sb, W1[:EMB], W1[EMB:],
               b1.reshape(1, -1), W2, b2.reshape(1, -1), W3.reshape(1, -1),
               b3.reshape(1, 1))
    return out.reshape(B)


# speedup vs baseline: 3.2337x; 3.2337x over previous
import functools

import jax
import jax.numpy as jnp
from jax import lax
from jax.experimental import pallas as pl
from jax.experimental.pallas import tpu as pltpu
from jax.experimental.pallas import tpu_sc as plsc

EMB = 64
WIDE = 2 * EMB
IDX_CHUNK = 128


PBLK = 16384


def _pair_body(a_ref, b_ref, out_ref):
    at = a_ref[...].astype(jnp.bfloat16).T.astype(jnp.float32)
    bt = b_ref[...].astype(jnp.bfloat16).T.astype(jnp.float32)
    out_ref[...] = jnp.concatenate([at, bt], axis=1)


def _pair_table(tt):
    n = tt.shape[1]
    nb = (n + 2 * PBLK - 1) // (2 * PBLK)
    last = (n - 1) // PBLK
    return pl.pallas_call(
        _pair_body,
        grid=(nb,),
        in_specs=[
            pl.BlockSpec((EMB, PBLK),
                         lambda i: (0, jnp.minimum(2 * i, last))),
            pl.BlockSpec((EMB, PBLK),
                         lambda i: (0, jnp.minimum(2 * i + 1, last))),
        ],
        out_specs=pl.BlockSpec((PBLK, WIDE), lambda i: (i, 0)),
        out_shape=jax.ShapeDtypeStruct((nb * PBLK, WIDE), jnp.float32),
    )(tt, tt)


def _make_gather(B):
    info = plsc.get_sparse_core_info()
    NC, NS = info.num_cores, info.num_subcores
    NW = NC * NS
    b_per_w = B // NW
    n_chunks = b_per_w // IDX_CHUNK
    mesh = plsc.VectorSubcoreMesh(core_axis_name="c", subcore_axis_name="s")

    @functools.partial(
        pl.kernel,
        mesh=mesh,
        out_type=[
            jax.ShapeDtypeStruct((B, WIDE), jnp.float32),
            jax.ShapeDtypeStruct((B, WIDE), jnp.float32),
        ],
        scratch_types=[
            pltpu.VMEM((b_per_w,), jnp.int32),
            pltpu.VMEM((b_per_w,), jnp.int32),
            pltpu.VMEM((2, IDX_CHUNK, WIDE), jnp.float32),
            pltpu.VMEM((2, IDX_CHUNK, WIDE), jnp.float32),
            pltpu.SemaphoreType.DMA,
            pltpu.SemaphoreType.DMA,
            pltpu.SemaphoreType.DMA,
            pltpu.SemaphoreType.DMA,
        ],
    )
    def gather_k(uidx_h, midx_h, ut_h, mt_h, uw_out, mw_out,
                 uidx_v, midx_v, ubuf, mbuf, su0, su1, sm0, sm1):
        wid = lax.axis_index("s") * NC + lax.axis_index("c")
        base = wid * b_per_w
        pltpu.sync_copy(uidx_h.at[pl.ds(base, b_per_w)], uidx_v)
        pltpu.sync_copy(midx_h.at[pl.ds(base, b_per_w)], midx_v)
        sems_u = (su0, su1)
        sems_m = (sm0, sm1)

        def start(c):
            slot = c % 2
            sl = pl.ds(c * IDX_CHUNK, IDX_CHUNK)
            cu = pltpu.async_copy(ut_h.at[uidx_v.at[sl]], ubuf.at[slot],
                                  sems_u[slot])
            cm = pltpu.async_copy(mt_h.at[midx_v.at[sl]], mbuf.at[slot],
                                  sems_m[slot])
            return cu, cm

        pend = start(0)
        for c in range(n_chunks):
            slot = c % 2
            cu, cm = pend
            if c + 1 < n_chunks:
                pend = start(c + 1)
            osl = pl.ds(base + c * IDX_CHUNK, IDX_CHUNK)
            cu.wait()
            pltpu.sync_copy(ubuf.at[slot], uw_out.at[osl])
            cm.wait()
            pltpu.sync_copy(mbuf.at[slot], mw_out.at[osl])

    return gather_k


def _mlp_body(uw_ref, mw_ref, pu_ref, pm_ref, w1u_ref, w1m_ref, b1_ref,
              w2_ref, b2_ref, w3_ref, b3_ref, out_ref):
    uw = uw_ref[...]
    mw = mw_ref[...]
    ue = uw[:, :EMB] + pu_ref[...] * (uw[:, EMB:] - uw[:, :EMB])
    me = mw[:, :EMB] + pm_ref[...] * (mw[:, EMB:] - mw[:, :EMB])
    h = jnp.dot(ue, w1u_ref[...], preferred_element_type=jnp.float32)
    h = h + jnp.dot(me, w1m_ref[...], preferred_element_type=jnp.float32)
    h = jnp.maximum(h + b1_ref[...], 0.0)
    h = jnp.dot(h, w2_ref[...], preferred_element_type=jnp.float32)
    h = jnp.maximum(h + b2_ref[...], 0.0)
    o = jnp.sum(h * w3_ref[...], axis=1) + b3_ref[0, 0]
    out_ref[...] = o[None, :]


def _mlp(uw, mw, pu, pm, w1u, w1m, b1, w2, b2, w3row, b3, blk=2048):
    B = uw.shape[0]
    grid = (B // blk,)
    const = lambda i: (0, 0)
    return pl.pallas_call(
        _mlp_body,
        grid=grid,
        in_specs=[
            pl.BlockSpec((blk, WIDE), lambda i: (i, 0)),
            pl.BlockSpec((blk, WIDE), lambda i: (i, 0)),
            pl.BlockSpec((blk, 1), lambda i: (i, 0)),
            pl.BlockSpec((blk, 1), lambda i: (i, 0)),
            pl.BlockSpec((EMB, 128), const),
            pl.BlockSpec((EMB, 128), const),
            pl.BlockSpec((1, 128), const),
            pl.BlockSpec((128, 64), const),
            pl.BlockSpec((1, 64), const),
            pl.BlockSpec((1, 64), const),
            pl.BlockSpec((1, 1), const),
        ],
        out_specs=pl.BlockSpec((1, blk), lambda i: (0, i)),
        out_shape=jax.ShapeDtypeStruct((1, B), jnp.float32),
    )(uw, mw, pu, pm, w1u, w1m, b1, w2, b2, w3row, b3)


def kernel(users, movies, user_table, movie_table, W1, b1, W2, b2, W3, b3):
    B = users.shape[0]
    nu = user_table.shape[0]
    nm = movie_table.shape[0]
    u32 = users.astype(jnp.int32)
    m32 = movies.astype(jnp.int32)
    ut2 = _pair_table(user_table.T)
    mt2 = jnp.concatenate(
        [movie_table[: nm // 2], movie_table[nm // 2 :]], axis=1)
    gather_k = _make_gather(B)
    urow = (u32 // (2 * PBLK)) * PBLK + (u32 & (PBLK - 1))
    uw, mw = gather_k(urow, m32 % (nm // 2), ut2, mt2)
    pu = ((u32 & (2 * PBLK - 1)) >= PBLK).astype(jnp.float32).reshape(B, 1)
    pm = (m32 >= nm // 2).astype(jnp.float32).reshape(B, 1)
    out = _mlp(uw, mw, pu, pm, W1[:EMB], W1[EMB:], b1.reshape(1, -1), W2,
               b2.reshape(1, -1), W3.reshape(1, -1), b3.reshape(1, 1))
    return out.reshape(B)


# speedup vs baseline: 3.4572x; 1.0691x over previous
import functools

import jax
import jax.numpy as jnp
from jax import lax
from jax.experimental import pallas as pl
from jax.experimental.pallas import tpu as pltpu
from jax.experimental.pallas import tpu_sc as plsc

EMB = 64
PACK = 2 * EMB
IDX_CHUNK = 128

G = 8192


def _quad_body(t_ref, out_ref):
    t = t_ref[...].astype(jnp.bfloat16).T
    y = jnp.concatenate([t[: 2 * G], t[2 * G:]], axis=1)
    out_ref[...] = pltpu.bitcast(y, jnp.float32)


def _quad_table(tt):
    n = tt.shape[1]
    nb = (n + 4 * G - 1) // (4 * G)
    return pl.pallas_call(
        _quad_body,
        grid=(nb,),
        in_specs=[pl.BlockSpec((EMB, 4 * G), lambda i: (0, i))],
        out_specs=pl.BlockSpec((G, PACK), lambda i: (i, 0)),
        out_shape=jax.ShapeDtypeStruct((nb * G, PACK), jnp.float32),
    )(tt)


def _make_gather(B):
    info = plsc.get_sparse_core_info()
    NC, NS = info.num_cores, info.num_subcores
    NW = NC * NS
    b_per_w = B // NW
    n_chunks = b_per_w // IDX_CHUNK
    mesh = plsc.VectorSubcoreMesh(core_axis_name="c", subcore_axis_name="s")

    @functools.partial(
        pl.kernel,
        mesh=mesh,
        out_type=[
            jax.ShapeDtypeStruct((B, PACK), jnp.float32),
            jax.ShapeDtypeStruct((B, PACK), jnp.float32),
        ],
        scratch_types=[
            pltpu.VMEM((b_per_w,), jnp.int32),
            pltpu.VMEM((b_per_w,), jnp.int32),
            pltpu.VMEM((2, IDX_CHUNK, PACK), jnp.float32),
            pltpu.VMEM((2, IDX_CHUNK, PACK), jnp.float32),
            pltpu.SemaphoreType.DMA,
            pltpu.SemaphoreType.DMA,
            pltpu.SemaphoreType.DMA,
            pltpu.SemaphoreType.DMA,
        ],
    )
    def gather_k(uidx_h, midx_h, ut_h, mt_h, uw_out, mw_out,
                 uidx_v, midx_v, ubuf, mbuf, su0, su1, sm0, sm1):
        wid = lax.axis_index("s") * NC + lax.axis_index("c")
        base = wid * b_per_w
        pltpu.sync_copy(uidx_h.at[pl.ds(base, b_per_w)], uidx_v)
        pltpu.sync_copy(midx_h.at[pl.ds(base, b_per_w)], midx_v)
        sems_u = (su0, su1)
        sems_m = (sm0, sm1)

        def start(c):
            slot = c % 2
            sl = pl.ds(c * IDX_CHUNK, IDX_CHUNK)
            cu = pltpu.async_copy(ut_h.at[uidx_v.at[sl]], ubuf.at[slot],
                                  sems_u[slot])
            cm = pltpu.async_copy(mt_h.at[midx_v.at[sl]], mbuf.at[slot],
                                  sems_m[slot])
            return cu, cm

        pend = start(0)
        for c in range(n_chunks):
            slot = c % 2
            cu, cm = pend
            if c + 1 < n_chunks:
                pend = start(c + 1)
            osl = pl.ds(base + c * IDX_CHUNK, IDX_CHUNK)
            cu.wait()
            pltpu.sync_copy(ubuf.at[slot], uw_out.at[osl])
            cm.wait()
            pltpu.sync_copy(mbuf.at[slot], mw_out.at[osl])

    return gather_k


def _unpack(w_ref, p_ref, h_ref):
    blk = w_ref.shape[0]
    x = pltpu.bitcast(w_ref[...], jnp.bfloat16)
    x = x.reshape(blk, 2, PACK).astype(jnp.float32)
    a = x[:, 0, :]
    b = x[:, 1, :]
    s = a + p_ref[...] * (b - a)
    return s[:, :EMB] + h_ref[...] * (s[:, EMB:] - s[:, :EMB])


def _mlp_body(uw_ref, mw_ref, pu_ref, hu_ref, pm_ref, hm_ref,
              w1u_ref, w1m_ref, b1_ref, w2_ref, b2_ref, w3_ref, b3_ref,
              out_ref):
    ue = _unpack(uw_ref, pu_ref, hu_ref)
    me = _unpack(mw_ref, pm_ref, hm_ref)
    h = jnp.dot(ue, w1u_ref[...], preferred_element_type=jnp.float32)
    h = h + jnp.dot(me, w1m_ref[...], preferred_element_type=jnp.float32)
    h = jnp.maximum(h + b1_ref[...], 0.0)
    h = jnp.dot(h, w2_ref[...], preferred_element_type=jnp.float32)
    h = jnp.maximum(h + b2_ref[...], 0.0)
    o = jnp.sum(h * w3_ref[...], axis=1) + b3_ref[0, 0]
    out_ref[...] = o[None, :]


def _mlp(uw, mw, pu, hu, pm, hm, w1u, w1m, b1, w2, b2, w3row, b3,
         blk=2048):
    B = uw.shape[0]
    grid = (B // blk,)
    const = lambda i: (0, 0)
    return pl.pallas_call(
        _mlp_body,
        grid=grid,
        in_specs=[
            pl.BlockSpec((blk, PACK), lambda i: (i, 0)),
            pl.BlockSpec((blk, PACK), lambda i: (i, 0)),
            pl.BlockSpec((blk, 1), lambda i: (i, 0)),
            pl.BlockSpec((blk, 1), lambda i: (i, 0)),
            pl.BlockSpec((blk, 1), lambda i: (i, 0)),
            pl.BlockSpec((blk, 1), lambda i: (i, 0)),
            pl.BlockSpec((EMB, 128), const),
            pl.BlockSpec((EMB, 128), const),
            pl.BlockSpec((1, 128), const),
            pl.BlockSpec((128, 64), const),
            pl.BlockSpec((1, 64), const),
            pl.BlockSpec((1, 64), const),
            pl.BlockSpec((1, 1), const),
        ],
        out_specs=pl.BlockSpec((1, blk), lambda i: (0, i)),
        out_shape=jax.ShapeDtypeStruct((1, B), jnp.float32),
    )(uw, mw, pu, hu, pm, hm, w1u, w1m, b1, w2, b2, w3row, b3)


def _quad_index(i32):
    row = (i32 // (4 * G)) * G + ((i32 & (2 * G - 1)) >> 1)
    p = (i32 & 1).astype(jnp.float32)
    h = ((i32 // (2 * G)) & 1).astype(jnp.float32)
    B = i32.shape[0]
    return row, p.reshape(B, 1), h.reshape(B, 1)


def kernel(users, movies, user_table, movie_table, W1, b1, W2, b2, W3, b3):
    B = users.shape[0]
    u32 = users.astype(jnp.int32)
    m32 = movies.astype(jnp.int32)
    ut4 = _quad_table(user_table.T)
    mt4 = _quad_table(movie_table.T)
    gather_k = _make_gather(B)
    urow, pu, hu = _quad_index(u32)
    mrow, pm, hm = _quad_index(m32)
    uw, mw = gather_k(urow, mrow, ut4, mt4)
    out = _mlp(uw, mw, pu, hu, pm, hm, W1[:EMB], W1[EMB:],
               b1.reshape(1, -1), W2, b2.reshape(1, -1), W3.reshape(1, -1),
               b3.reshape(1, 1))
    return out.reshape(B)
